# baseline (speedup 1.0000x reference)
import functools

import numpy as np
import jax
import jax.numpy as jnp
from jax import lax
from jax.experimental import pallas as pl
from jax.experimental.pallas import tpu as pltpu
from jax.experimental.pallas import tpu_sc as plsc

N = 10000
E = 160000
D = 256
MD = 128
NR = 32
NB = 4
G = 32
NT = 101
OUT = 10
CUTOFF = 5.0
P = 6

BN = 1000
NSTEP = N // BN
E_PAD = 163840
BE = 2048
ESTEP = E_PAD // BE
CE = 64
NCHUNK = E_PAD // CE
NW = 32
CPW = NCHUNK // NW
NPAD = 10112
RPT = NPAD // 16

_sc_mesh = plsc.VectorSubcoreMesh(core_axis_name="c", subcore_axis_name="s")


@functools.partial(
    pl.kernel,
    mesh=_sc_mesh,
    out_type=jax.ShapeDtypeStruct((NCHUNK, CE), jnp.float32),
    scratch_types=[
        pltpu.VMEM((3 * N,), jnp.float32),
        pltpu.VMEM((CPW, CE), jnp.int32),
        pltpu.VMEM((CPW, CE), jnp.int32),
        pltpu.VMEM((CPW, CE), jnp.float32),
    ],
    compiler_params=pltpu.CompilerParams(needs_layout_passes=False),
)
def _dist_kernel(pos_hbm, src_hbm, dst_hbm, d2_hbm, pos_v, srcb, dstb, outb):
    cid = lax.axis_index("c")
    sid = lax.axis_index("s")
    w = sid * 2 + cid
    base = w * CPW
    pltpu.sync_copy(pos_hbm, pos_v)
    pltpu.sync_copy(src_hbm.at[pl.ds(base, CPW)], srcb)
    pltpu.sync_copy(dst_hbm.at[pl.ds(base, CPW)], dstb)

    def chunk(i, carry):
        for g in range(CE // 16):
            sl = pl.ds(g * 16, 16)
            si = srcb[i, sl] * 3
            di = dstb[i, sl] * 3
            dx = plsc.load_gather(pos_v, [si]) - plsc.load_gather(pos_v, [di])
            dy = plsc.load_gather(pos_v, [si + 1]) - plsc.load_gather(pos_v, [di + 1])
            dz = plsc.load_gather(pos_v, [si + 2]) - plsc.load_gather(pos_v, [di + 2])
            outb[i, sl] = dx * dx + dy * dy + dz * dz
        return carry

    lax.fori_loop(0, CPW, chunk, 0)
    pltpu.sync_copy(outb, d2_hbm.at[pl.ds(base, CPW)])


PAIR = NCHUNK // 16
HP = 40
NPH0 = 4
A0 = NPH0 * HP


@functools.partial(
    pl.kernel,
    mesh=_sc_mesh,
    out_type=jax.ShapeDtypeStruct((2, NPAD, MD), jnp.float32),
    scratch_types=[
        pltpu.VMEM_SHARED((NPAD, MD), jnp.float32),
        pltpu.VMEM((HP, CE), jnp.int32),
        pltpu.VMEM((HP, CE), jnp.int32),
        pltpu.VMEM((CE, MD), jnp.float32),
        pltpu.VMEM((CE, MD), jnp.float32),
        pltpu.VMEM((CE, MD), jnp.float32),
        pltpu.VMEM((CE, MD), jnp.float32),
        pltpu.SemaphoreType.DMA,
        pltpu.SemaphoreType.DMA,
        pltpu.SemaphoreType.DMA,
        pltpu.SemaphoreType.DMA,
        pltpu.SemaphoreType.DMA,
        pltpu.SemaphoreType.DMA,
    ],
    compiler_params=pltpu.CompilerParams(needs_layout_passes=False),
)
def _conv_kernel(hw_hbm, filt_hbm, src_hbm, dst_hbm, zeros_hbm, out_hbm,
                 acc, srcb, dstb, gb0, gb1, fb0, fb1,
                 gsem0, gsem1, fsem0, fsem1, ssem0, ssem1):
    cid = lax.axis_index("c")
    sid = lax.axis_index("s")
    gbufs = (gb0, gb1)
    fbufs = (fb0, fb1)
    gsems = (gsem0, gsem1)
    fsems = (fsem0, fsem1)
    ssems = (ssem0, ssem1)

    pltpu.sync_copy(zeros_hbm.at[pl.ds(sid * RPT, RPT)],
                    acc.at[pl.ds(sid * RPT, RPT)])
    plsc.subcore_barrier()

    base = sid * PAIR + cid * A0
    nph = jnp.where(cid == 0, NPH0, (PAIR - A0) // HP)

    def phase(p, pcarry):
        pbase = base + p * HP
        pltpu.sync_copy(src_hbm.at[pl.ds(pbase, HP)], srcb)
        pltpu.sync_copy(dst_hbm.at[pl.ds(pbase, HP)], dstb)

        def start(b, c):
            pltpu.async_copy(hw_hbm.at[dstb.at[c]], gbufs[b], gsems[b])
            pltpu.async_copy(filt_hbm.at[pl.ds((pbase + c) * CE, CE)],
                             fbufs[b], fsems[b])

        for b in range(2):
            start(b, b)

        def body(i, carry):
            for b in range(2):
                c = 2 * i + b
                pltpu.make_async_copy(hw_hbm.at[dstb.at[c]], gbufs[b],
                                      gsems[b]).wait()
                pltpu.make_async_copy(
                    filt_hbm.at[pl.ds((pbase + c) * CE, CE)],
                    fbufs[b], fsems[b]).wait()

                def mul_row(e, cr, _gb=gbufs[b], _fb=fbufs[b]):
                    for j in range(MD // 16):
                        sl = pl.ds(j * 16, 16)
                        _gb[e, sl] = _gb[e, sl] * _fb[e, sl]
                    return cr

                lax.fori_loop(0, CE, mul_row, 0)
                pltpu.async_copy(gbufs[b], acc.at[srcb.at[c]], ssems[b],
                                 add=True)
            for b in range(2):
                c = 2 * i + b
                pltpu.make_async_copy(gbufs[b], acc.at[srcb.at[c]],
                                      ssems[b]).wait()

                @pl.when(c + 2 < HP)
                def _(b=b, c=c):
                    start(b, c + 2)

            return carry

        lax.fori_loop(0, HP // 2, body, 0)
        return pcarry

    lax.fori_loop(0, nph, phase, 0)

    plsc.subcore_barrier()
    pltpu.sync_copy(acc.at[pl.ds(sid * RPT, RPT)],
                    out_hbm.at[cid, pl.ds(sid * RPT, RPT)])


def _embed_body(x_ref, tab_ref, embw_ref, embb_ref, win0_ref, h_ref, hw_ref):
    xi = x_ref[...]
    fused = jnp.dot(tab_ref[...], embw_ref[...],
                    preferred_element_type=jnp.float32)
    oh = (lax.broadcasted_iota(jnp.int32, (BN, NT), 1) == xi).astype(jnp.float32)
    h0 = jax.nn.gelu(jnp.dot(oh, fused, preferred_element_type=jnp.float32)
                     + embb_ref[...])
    h_ref[...] = h0
    hw_ref[...] = jnp.dot(h0, win0_ref[...], preferred_element_type=jnp.float32)


_embed_call = pl.pallas_call(
    _embed_body,
    grid=(NSTEP,),
    in_specs=[
        pl.BlockSpec((BN, 1), lambda i: (i, 0)),
        pl.BlockSpec((NT, 5), lambda i: (0, 0)),
        pl.BlockSpec((5, D), lambda i: (0, 0)),
        pl.BlockSpec((1, D), lambda i: (0, 0)),
        pl.BlockSpec((D, MD), lambda i: (0, 0)),
    ],
    out_specs=[
        pl.BlockSpec((BN, D), lambda i: (i, 0)),
        pl.BlockSpec((BN, MD), lambda i: (i, 0)),
    ],
    out_shape=[
        jax.ShapeDtypeStruct((N, D), jnp.float32),
        jax.ShapeDtypeStruct((N, MD), jnp.float32),
    ],
)


def _filt_body(d2_ref, fw_ref, fb_ref, *out_refs):
    d2 = d2_ref[...]
    dist = jnp.sqrt(d2 + 1e-12)
    d = jnp.maximum(dist / CUTOFF, 1e-6)
    dsq = d * d
    d4 = dsq * dsq
    d5 = d4 * d
    a = -(P + 1) * (P + 2) / 2.0
    b = P * (P + 2)
    c = -P * (P + 1) / 2.0
    env = 1.0 / d + a * d5 + b * d5 * d + c * d5 * dsq
    env = jnp.where(dist < CUTOFF, env, 0.0) * np.float32(np.sqrt(2.0 / CUTOFF))
    freq = (lax.broadcasted_iota(jnp.int32, (1, NR), 1).astype(jnp.float32)
            + 1.0) * np.float32(np.pi)
    rbf = env * jnp.sin(d * freq)
    for blk in range(NB):
        out_refs[blk][...] = (
            jnp.dot(rbf, fw_ref[blk], preferred_element_type=jnp.float32)
            + fb_ref[blk])


_filt_call = pl.pallas_call(
    _filt_body,
    grid=(ESTEP,),
    in_specs=[
        pl.BlockSpec((BE, 1), lambda i: (i, 0)),
        pl.BlockSpec((NB, NR, MD), lambda i: (0, 0, 0)),
        pl.BlockSpec((NB, 1, MD), lambda i: (0, 0, 0)),
    ],
    out_specs=[pl.BlockSpec((BE, MD), lambda i: (i, 0)) for _ in range(NB)],
    out_shape=[jax.ShapeDtypeStruct((E_PAD, MD), jnp.float32)
               for _ in range(NB)],
)


def _block_body(parts_ref, h_ref, wout_ref, bout_ref, w1_ref, b1_ref,
                w2_ref, b2_ref, winn_ref, h_out_ref, hw_out_ref):
    agg = parts_ref[0] + parts_ref[1]
    t = jax.nn.gelu(jnp.dot(agg, wout_ref[...],
                            preferred_element_type=jnp.float32) + bout_ref[...])
    h1 = h_ref[...] + t
    u = jax.nn.gelu(jnp.dot(h1, w1_ref[...],
                            preferred_element_type=jnp.float32) + b1_ref[...])
    h2 = h1 + jnp.dot(u, w2_ref[...], preferred_element_type=jnp.float32) + b2_ref[...]
    h_out_ref[...] = h2
    hw_out_ref[...] = jnp.dot(h2, winn_ref[...], preferred_element_type=jnp.float32)


_block_call = pl.pallas_call(
    _block_body,
    grid=(NSTEP,),
    in_specs=[
        pl.BlockSpec((2, BN, MD), lambda i: (0, i, 0)),
        pl.BlockSpec((BN, D), lambda i: (i, 0)),
        pl.BlockSpec((MD, D), lambda i: (0, 0)),
        pl.BlockSpec((1, D), lambda i: (0, 0)),
        pl.BlockSpec((D, D), lambda i: (0, 0)),
        pl.BlockSpec((1, D), lambda i: (0, 0)),
        pl.BlockSpec((D, D), lambda i: (0, 0)),
        pl.BlockSpec((1, D), lambda i: (0, 0)),
        pl.BlockSpec((D, MD), lambda i: (0, 0)),
    ],
    out_specs=[
        pl.BlockSpec((BN, D), lambda i: (i, 0)),
        pl.BlockSpec((BN, MD), lambda i: (i, 0)),
    ],
    out_shape=[
        jax.ShapeDtypeStruct((N, D), jnp.float32),
        jax.ShapeDtypeStruct((N, MD), jnp.float32),
    ],
)


def _pool_body(b3_ref, h_ref, hw1_ref, hb1_ref, hw2_ref, hb2_ref, out_ref,
               gsum, cnt):
    i = pl.program_id(0)

    @pl.when(i == 0)
    def _():
        gsum[...] = jnp.zeros_like(gsum)
        cnt[...] = jnp.zeros_like(cnt)

    bb = jnp.minimum(b3_ref[0], G - 1)
    oh = (lax.broadcasted_iota(jnp.int32, (G, BN), 0) == bb).astype(jnp.float32)
    gsum[...] += jnp.dot(oh, h_ref[...], preferred_element_type=jnp.float32)
    cnt[...] += jnp.sum(oh, axis=1, keepdims=True)

    @pl.when(i == NSTEP - 1)
    def _():
        g = gsum[...] / jnp.maximum(cnt[...], 1.0)
        z = jax.nn.gelu(jnp.dot(g, hw1_ref[...],
                                preferred_element_type=jnp.float32) + hb1_ref[...])
        out_ref[...] = jnp.dot(z, hw2_ref[...],
                               preferred_element_type=jnp.float32) + hb2_ref[...]


_pool_call = pl.pallas_call(
    _pool_body,
    grid=(NSTEP,),
    in_specs=[
        pl.BlockSpec((1, 1, BN), lambda i: (i, 0, 0)),
        pl.BlockSpec((BN, D), lambda i: (i, 0)),
        pl.BlockSpec((D, D), lambda i: (0, 0)),
        pl.BlockSpec((1, D), lambda i: (0, 0)),
        pl.BlockSpec((D, OUT), lambda i: (0, 0)),
        pl.BlockSpec((1, OUT), lambda i: (0, 0)),
    ],
    out_specs=pl.BlockSpec((G, OUT), lambda i: (0, 0)),
    out_shape=jax.ShapeDtypeStruct((G, OUT), jnp.float32),
    scratch_shapes=[
        pltpu.VMEM((G, D), jnp.float32),
        pltpu.VMEM((G, 1), jnp.float32),
    ],
)


def kernel(x, pos, edge_index, batch, num_graphs, emb_table, emb_W, emb_b,
           W_in, filt_W, filt_b, W_out, b_out, fc_W1, fc_b1, fc_W2, fc_b2,
           head_W1, head_b1, head_W2, head_b2):
    del num_graphs
    pos_flat = pos.reshape(-1)
    src = edge_index[0].astype(jnp.int32)
    dst = edge_index[1].astype(jnp.int32)
    pad = E_PAD - E
    src_dist = jnp.pad(src, (0, pad)).reshape(NCHUNK, CE)
    dst_pad = jnp.pad(dst, (0, pad)).reshape(NCHUNK, CE)
    src_scat = jnp.pad(src, (0, pad), constant_values=N).reshape(NCHUNK, CE)

    d2 = _dist_kernel(pos_flat, src_dist, dst_pad)
    filts = _filt_call(d2.reshape(E_PAD, 1), filt_W, filt_b.reshape(NB, 1, MD))
    h, hw = _embed_call(x, emb_table, emb_W, emb_b.reshape(1, D), W_in[0])

    zeros = jnp.zeros((NPAD, MD), jnp.float32)
    for blk in range(NB):
        parts = _conv_kernel(hw, filts[blk], src_scat, dst_pad, zeros)
        h, hw = _block_call(parts, h,
                            W_out[blk], b_out[blk].reshape(1, D),
                            fc_W1[blk], fc_b1[blk].reshape(1, D),
                            fc_W2[blk], fc_b2[blk].reshape(1, D),
                            W_in[(blk + 1) % NB])

    return _pool_call(batch.reshape(NSTEP, 1, BN), h,
                      head_W1, head_b1.reshape(1, D),
                      head_W2, head_b2.reshape(1, OUT))


# speedup vs baseline: 1.1138x; 1.1138x over previous
import functools

import numpy as np
import jax
import jax.numpy as jnp
from jax import lax
from jax.experimental import pallas as pl
from jax.experimental.pallas import tpu as pltpu
from jax.experimental.pallas import tpu_sc as plsc

N = 10000
E = 160000
D = 256
MD = 128
NR = 32
NB = 4
G = 32
NT = 101
OUT = 10
CUTOFF = 5.0
P = 6

BN = 1000
NSTEP = N // BN
E_PAD = 163840
BE = 2048
ESTEP = E_PAD // BE
CE = 64
NCHUNK = E_PAD // CE
NW = 32
CPW = NCHUNK // NW
NPAD = 10112
RPT = NPAD // 16

_sc_mesh = plsc.VectorSubcoreMesh(core_axis_name="c", subcore_axis_name="s")


@functools.partial(
    pl.kernel,
    mesh=_sc_mesh,
    out_type=jax.ShapeDtypeStruct((NCHUNK, CE), jnp.float32),
    scratch_types=[
        pltpu.VMEM((3 * N,), jnp.float32),
        pltpu.VMEM((CPW, CE), jnp.int32),
        pltpu.VMEM((CPW, CE), jnp.int32),
        pltpu.VMEM((CPW, CE), jnp.float32),
    ],
    compiler_params=pltpu.CompilerParams(needs_layout_passes=False),
)
def _dist_kernel(pos_hbm, src_hbm, dst_hbm, d2_hbm, pos_v, srcb, dstb, outb):
    cid = lax.axis_index("c")
    sid = lax.axis_index("s")
    w = sid * 2 + cid
    base = w * CPW
    pltpu.sync_copy(pos_hbm, pos_v)
    pltpu.sync_copy(src_hbm.at[pl.ds(base, CPW)], srcb)
    pltpu.sync_copy(dst_hbm.at[pl.ds(base, CPW)], dstb)

    def chunk(i, carry):
        for g in range(CE // 16):
            sl = pl.ds(g * 16, 16)
            si = srcb[i, sl] * 3
            di = dstb[i, sl] * 3
            dx = plsc.load_gather(pos_v, [si]) - plsc.load_gather(pos_v, [di])
            dy = plsc.load_gather(pos_v, [si + 1]) - plsc.load_gather(pos_v, [di + 1])
            dz = plsc.load_gather(pos_v, [si + 2]) - plsc.load_gather(pos_v, [di + 2])
            outb[i, sl] = dx * dx + dy * dy + dz * dz
        return carry

    lax.fori_loop(0, CPW, chunk, 0)
    pltpu.sync_copy(outb, d2_hbm.at[pl.ds(base, CPW)])


PAIR = NCHUNK // 16
HP = 16
NPH0 = 9
A0 = NPH0 * HP


@functools.partial(
    pl.kernel,
    mesh=_sc_mesh,
    out_type=jax.ShapeDtypeStruct((2, NPAD, MD), jnp.float32),
    scratch_types=[
        pltpu.VMEM_SHARED((NPAD, MD), jnp.float32),
        pltpu.VMEM((HP, CE), jnp.int32),
        pltpu.VMEM((HP, CE), jnp.int32),
        pltpu.VMEM((CE, MD), jnp.float32),
        pltpu.VMEM((CE, MD), jnp.float32),
        pltpu.VMEM((CE, MD), jnp.float32),
        pltpu.VMEM((CE, MD), jnp.float32),
        pltpu.SemaphoreType.DMA,
        pltpu.SemaphoreType.DMA,
        pltpu.SemaphoreType.DMA,
        pltpu.SemaphoreType.DMA,
        pltpu.SemaphoreType.DMA,
        pltpu.SemaphoreType.DMA,
    ],
    compiler_params=pltpu.CompilerParams(needs_layout_passes=False),
)
def _conv_kernel(hw_hbm, filt_hbm, src_hbm, dst_hbm, zeros_hbm, out_hbm,
                 acc, srcb, dstb, gb0, gb1, fb0, fb1,
                 gsem0, gsem1, fsem0, fsem1, ssem0, ssem1):
    cid = lax.axis_index("c")
    sid = lax.axis_index("s")
    gbufs = (gb0, gb1)
    fbufs = (fb0, fb1)
    gsems = (gsem0, gsem1)
    fsems = (fsem0, fsem1)
    ssems = (ssem0, ssem1)

    pltpu.sync_copy(zeros_hbm.at[pl.ds(sid * RPT, RPT)],
                    acc.at[pl.ds(sid * RPT, RPT)])
    plsc.subcore_barrier()

    base = sid * PAIR + cid * A0
    nph = jnp.where(cid == 0, NPH0, (PAIR - A0) // HP)

    def phase(p, pcarry):
        pbase = base + p * HP
        pltpu.sync_copy(src_hbm.at[pl.ds(pbase, HP)], srcb)
        pltpu.sync_copy(dst_hbm.at[pl.ds(pbase, HP)], dstb)

        def start(b, c):
            pltpu.async_copy(hw_hbm.at[dstb.at[c]], gbufs[b], gsems[b])
            pltpu.async_copy(filt_hbm.at[pl.ds((pbase + c) * CE, CE)],
                             fbufs[b], fsems[b])

        for b in range(2):
            start(b, b)

        def body(i, carry):
            for b in range(2):
                c = 2 * i + b
                pltpu.make_async_copy(hw_hbm.at[dstb.at[c]], gbufs[b],
                                      gsems[b]).wait()
                pltpu.make_async_copy(
                    filt_hbm.at[pl.ds((pbase + c) * CE, CE)],
                    fbufs[b], fsems[b]).wait()

                def mul_row(e, cr, _gb=gbufs[b], _fb=fbufs[b]):
                    for j in range(MD // 16):
                        sl = pl.ds(j * 16, 16)
                        _gb[e, sl] = _gb[e, sl] * _fb[e, sl]
                    return cr

                lax.fori_loop(0, CE, mul_row, 0)
                pltpu.async_copy(gbufs[b], acc.at[srcb.at[c]], ssems[b],
                                 add=True)
            for b in range(2):
                c = 2 * i + b
                pltpu.make_async_copy(gbufs[b], acc.at[srcb.at[c]],
                                      ssems[b]).wait()

                @pl.when(c + 2 < HP)
                def _(b=b, c=c):
                    start(b, c + 2)

            return carry

        lax.fori_loop(0, HP // 2, body, 0)
        return pcarry

    lax.fori_loop(0, nph, phase, 0)

    plsc.subcore_barrier()
    pltpu.sync_copy(acc.at[pl.ds(sid * RPT, RPT)],
                    out_hbm.at[cid, pl.ds(sid * RPT, RPT)])


def _embed_body(x_ref, tab_ref, embw_ref, embb_ref, win0_ref, h_ref, hw_ref):
    xi = x_ref[...]
    fused = jnp.dot(tab_ref[...], embw_ref[...],
                    preferred_element_type=jnp.float32)
    oh = (lax.broadcasted_iota(jnp.int32, (BN, NT), 1) == xi).astype(jnp.float32)
    h0 = jax.nn.gelu(jnp.dot(oh, fused, preferred_element_type=jnp.float32)
                     + embb_ref[...])
    h_ref[...] = h0
    hw_ref[...] = jnp.dot(h0, win0_ref[...], preferred_element_type=jnp.float32)


_embed_call = pl.pallas_call(
    _embed_body,
    grid=(NSTEP,),
    in_specs=[
        pl.BlockSpec((BN, 1), lambda i: (i, 0)),
        pl.BlockSpec((NT, 5), lambda i: (0, 0)),
        pl.BlockSpec((5, D), lambda i: (0, 0)),
        pl.BlockSpec((1, D), lambda i: (0, 0)),
        pl.BlockSpec((D, MD), lambda i: (0, 0)),
    ],
    out_specs=[
        pl.BlockSpec((BN, D), lambda i: (i, 0)),
        pl.BlockSpec((BN, MD), lambda i: (i, 0)),
    ],
    out_shape=[
        jax.ShapeDtypeStruct((N, D), jnp.float32),
        jax.ShapeDtypeStruct((N, MD), jnp.float32),
    ],
)


def _filt_body(d2_ref, fw_ref, fb_ref, *out_refs):
    d2 = d2_ref[...]
    dist = jnp.sqrt(d2 + 1e-12)
    d = jnp.maximum(dist / CUTOFF, 1e-6)
    dsq = d * d
    d4 = dsq * dsq
    d5 = d4 * d
    a = -(P + 1) * (P + 2) / 2.0
    b = P * (P + 2)
    c = -P * (P + 1) / 2.0
    env = 1.0 / d + a * d5 + b * d5 * d + c * d5 * dsq
    env = jnp.where(dist < CUTOFF, env, 0.0) * np.float32(np.sqrt(2.0 / CUTOFF))
    freq = (lax.broadcasted_iota(jnp.int32, (1, NR), 1).astype(jnp.float32)
            + 1.0) * np.float32(np.pi)
    rbf = env * jnp.sin(d * freq)
    for blk in range(NB):
        out_refs[blk][...] = (
            jnp.dot(rbf, fw_ref[blk], preferred_element_type=jnp.float32)
            + fb_ref[blk])


_filt_call = pl.pallas_call(
    _filt_body,
    grid=(ESTEP,),
    in_specs=[
        pl.BlockSpec((BE, 1), lambda i: (i, 0)),
        pl.BlockSpec((NB, NR, MD), lambda i: (0, 0, 0)),
        pl.BlockSpec((NB, 1, MD), lambda i: (0, 0, 0)),
    ],
    out_specs=[pl.BlockSpec((BE, MD), lambda i: (i, 0)) for _ in range(NB)],
    out_shape=[jax.ShapeDtypeStruct((E_PAD, MD), jnp.float32)
               for _ in range(NB)],
)


def _block_body(parts_ref, h_ref, wout_ref, bout_ref, w1_ref, b1_ref,
                w2_ref, b2_ref, winn_ref, h_out_ref, hw_out_ref):
    agg = parts_ref[0] + parts_ref[1]
    t = jax.nn.gelu(jnp.dot(agg, wout_ref[...],
                            preferred_element_type=jnp.float32) + bout_ref[...])
    h1 = h_ref[...] + t
    u = jax.nn.gelu(jnp.dot(h1, w1_ref[...],
                            preferred_element_type=jnp.float32) + b1_ref[...])
    h2 = h1 + jnp.dot(u, w2_ref[...], preferred_element_type=jnp.float32) + b2_ref[...]
    h_out_ref[...] = h2
    hw_out_ref[...] = jnp.dot(h2, winn_ref[...], preferred_element_type=jnp.float32)


_block_call = pl.pallas_call(
    _block_body,
    grid=(NSTEP,),
    in_specs=[
        pl.BlockSpec((2, BN, MD), lambda i: (0, i, 0)),
        pl.BlockSpec((BN, D), lambda i: (i, 0)),
        pl.BlockSpec((MD, D), lambda i: (0, 0)),
        pl.BlockSpec((1, D), lambda i: (0, 0)),
        pl.BlockSpec((D, D), lambda i: (0, 0)),
        pl.BlockSpec((1, D), lambda i: (0, 0)),
        pl.BlockSpec((D, D), lambda i: (0, 0)),
        pl.BlockSpec((1, D), lambda i: (0, 0)),
        pl.BlockSpec((D, MD), lambda i: (0, 0)),
    ],
    out_specs=[
        pl.BlockSpec((BN, D), lambda i: (i, 0)),
        pl.BlockSpec((BN, MD), lambda i: (i, 0)),
    ],
    out_shape=[
        jax.ShapeDtypeStruct((N, D), jnp.float32),
        jax.ShapeDtypeStruct((N, MD), jnp.float32),
    ],
)


def _pool_body(b3_ref, h_ref, hw1_ref, hb1_ref, hw2_ref, hb2_ref, out_ref,
               gsum, cnt):
    i = pl.program_id(0)

    @pl.when(i == 0)
    def _():
        gsum[...] = jnp.zeros_like(gsum)
        cnt[...] = jnp.zeros_like(cnt)

    bb = jnp.minimum(b3_ref[0], G - 1)
    oh = (lax.broadcasted_iota(jnp.int32, (G, BN), 0) == bb).astype(jnp.float32)
    gsum[...] += jnp.dot(oh, h_ref[...], preferred_element_type=jnp.float32)
    cnt[...] += jnp.sum(oh, axis=1, keepdims=True)

    @pl.when(i == NSTEP - 1)
    def _():
        g = gsum[...] / jnp.maximum(cnt[...], 1.0)
        z = jax.nn.gelu(jnp.dot(g, hw1_ref[...],
                                preferred_element_type=jnp.float32) + hb1_ref[...])
        out_ref[...] = jnp.dot(z, hw2_ref[...],
                               preferred_element_type=jnp.float32) + hb2_ref[...]


_pool_call = pl.pallas_call(
    _pool_body,
    grid=(NSTEP,),
    in_specs=[
        pl.BlockSpec((1, 1, BN), lambda i: (i, 0, 0)),
        pl.BlockSpec((BN, D), lambda i: (i, 0)),
        pl.BlockSpec((D, D), lambda i: (0, 0)),
        pl.BlockSpec((1, D), lambda i: (0, 0)),
        pl.BlockSpec((D, OUT), lambda i: (0, 0)),
        pl.BlockSpec((1, OUT), lambda i: (0, 0)),
    ],
    out_specs=pl.BlockSpec((G, OUT), lambda i: (0, 0)),
    out_shape=jax.ShapeDtypeStruct((G, OUT), jnp.float32),
    scratch_shapes=[
        pltpu.VMEM((G, D), jnp.float32),
        pltpu.VMEM((G, 1), jnp.float32),
    ],
)


def kernel(x, pos, edge_index, batch, num_graphs, emb_table, emb_W, emb_b,
           W_in, filt_W, filt_b, W_out, b_out, fc_W1, fc_b1, fc_W2, fc_b2,
           head_W1, head_b1, head_W2, head_b2):
    del num_graphs
    pos_flat = pos.reshape(-1)
    src = edge_index[0].astype(jnp.int32)
    dst = edge_index[1].astype(jnp.int32)
    pad = E_PAD - E
    src_dist = jnp.pad(src, (0, pad)).reshape(NCHUNK, CE)
    dst_pad = jnp.pad(dst, (0, pad)).reshape(NCHUNK, CE)
    src_scat = jnp.pad(src, (0, pad), constant_values=N).reshape(NCHUNK, CE)

    d2 = _dist_kernel(pos_flat, src_dist, dst_pad)
    filts = _filt_call(d2.reshape(E_PAD, 1), filt_W, filt_b.reshape(NB, 1, MD))
    h, hw = _embed_call(x, emb_table, emb_W, emb_b.reshape(1, D), W_in[0])

    zeros = jnp.zeros((NPAD, MD), jnp.float32)
    for blk in range(NB):
        parts = _conv_kernel(hw, filts[blk], src_scat, dst_pad, zeros)
        h, hw = _block_call(parts, h,
                            W_out[blk], b_out[blk].reshape(1, D),
                            fc_W1[blk], fc_b1[blk].reshape(1, D),
                            fc_W2[blk], fc_b2[blk].reshape(1, D),
                            W_in[(blk + 1) % NB])

    return _pool_call(batch.reshape(NSTEP, 1, BN), h,
                      head_W1, head_b1.reshape(1, D),
                      head_W2, head_b2.reshape(1, OUT))


# speedup vs baseline: 1.3837x; 1.2423x over previous
import functools

import numpy as np
import jax
import jax.numpy as jnp
from jax import lax
from jax.experimental import pallas as pl
from jax.experimental.pallas import tpu as pltpu
from jax.experimental.pallas import tpu_sc as plsc

N = 10000
E = 160000
D = 256
MD = 128
NR = 32
NB = 4
G = 32
NT = 101
OUT = 10
CUTOFF = 5.0
P = 6

BN = 1000
NSTEP = N // BN
E_PAD = 163840
BE = 4096
ESTEP = E_PAD // BE
CE = 64
NCHUNK = E_PAD // CE
NW = 32
CPW = NCHUNK // NW
NPAD = 10112
RPT = NPAD // 16

_sc_mesh = plsc.VectorSubcoreMesh(core_axis_name="c", subcore_axis_name="s")


@functools.partial(
    pl.kernel,
    mesh=_sc_mesh,
    out_type=jax.ShapeDtypeStruct((NCHUNK, CE), jnp.float32),
    scratch_types=[
        pltpu.VMEM((3 * N,), jnp.float32),
        pltpu.VMEM((CPW, CE), jnp.int32),
        pltpu.VMEM((CPW, CE), jnp.int32),
        pltpu.VMEM((CPW, CE), jnp.float32),
    ],
    compiler_params=pltpu.CompilerParams(needs_layout_passes=False),
)
def _dist_kernel(pos_hbm, src_hbm, dst_hbm, d2_hbm, pos_v, srcb, dstb, outb):
    cid = lax.axis_index("c")
    sid = lax.axis_index("s")
    w = sid * 2 + cid
    base = w * CPW
    pltpu.sync_copy(pos_hbm, pos_v)
    pltpu.sync_copy(src_hbm.at[pl.ds(base, CPW)], srcb)
    pltpu.sync_copy(dst_hbm.at[pl.ds(base, CPW)], dstb)

    def chunk(i, carry):
        for g in range(CE // 16):
            sl = pl.ds(g * 16, 16)
            si = srcb[i, sl] * 3
            di = dstb[i, sl] * 3
            dx = plsc.load_gather(pos_v, [si]) - plsc.load_gather(pos_v, [di])
            dy = plsc.load_gather(pos_v, [si + 1]) - plsc.load_gather(pos_v, [di + 1])
            dz = plsc.load_gather(pos_v, [si + 2]) - plsc.load_gather(pos_v, [di + 2])
            outb[i, sl] = dx * dx + dy * dy + dz * dz
        return carry

    lax.fori_loop(0, CPW, chunk, 0)
    pltpu.sync_copy(outb, d2_hbm.at[pl.ds(base, CPW)])


PAIR = NCHUNK // 16
HP = 40
NPH0 = 3
A0 = NPH0 * HP


@functools.partial(
    pl.kernel,
    mesh=_sc_mesh,
    out_type=jax.ShapeDtypeStruct((2, NPAD, MD), jnp.float32),
    scratch_types=[
        pltpu.VMEM_SHARED((NPAD, MD), jnp.float32),
        pltpu.VMEM((HP, CE), jnp.int32),
        pltpu.VMEM((HP, CE), jnp.int32),
        pltpu.VMEM((CE, MD), jnp.float32),
        pltpu.VMEM((CE, MD), jnp.float32),
        pltpu.VMEM((CE, MD), jnp.float32),
        pltpu.VMEM((CE, MD), jnp.float32),
        pltpu.SemaphoreType.DMA,
        pltpu.SemaphoreType.DMA,
        pltpu.SemaphoreType.DMA,
        pltpu.SemaphoreType.DMA,
        pltpu.SemaphoreType.DMA,
        pltpu.SemaphoreType.DMA,
    ],
    compiler_params=pltpu.CompilerParams(needs_layout_passes=False),
)
def _conv_kernel(hw_hbm, filt_hbm, src_hbm, dst_hbm, zeros_hbm, out_hbm,
                 acc, srcb, dstb, gb0, gb1, fb0, fb1,
                 gsem0, gsem1, fsem0, fsem1, ssem0, ssem1):
    cid = lax.axis_index("c")
    sid = lax.axis_index("s")
    gbufs = (gb0, gb1)
    fbufs = (fb0, fb1)
    gsems = (gsem0, gsem1)
    fsems = (fsem0, fsem1)
    ssems = (ssem0, ssem1)

    pltpu.sync_copy(zeros_hbm.at[pl.ds(sid * RPT, RPT)],
                    acc.at[pl.ds(sid * RPT, RPT)])
    plsc.subcore_barrier()

    base = sid * PAIR + cid * A0
    nph = jnp.where(cid == 0, NPH0, (PAIR - A0) // HP)

    def phase(p, pcarry):
        pbase = base + p * HP
        pltpu.sync_copy(src_hbm.at[pl.ds(pbase, HP)], srcb)
        pltpu.sync_copy(dst_hbm.at[pl.ds(pbase, HP)], dstb)

        def start(b, c):
            pltpu.async_copy(hw_hbm.at[dstb.at[c]], gbufs[b], gsems[b])
            pltpu.async_copy(filt_hbm.at[pl.ds((pbase + c) * CE, CE)],
                             fbufs[b], fsems[b])

        for b in range(2):
            start(b, b)

        def body(i, carry):
            for b in range(2):
                c = 2 * i + b
                pltpu.make_async_copy(hw_hbm.at[dstb.at[c]], gbufs[b],
                                      gsems[b]).wait()
                pltpu.make_async_copy(
                    filt_hbm.at[pl.ds((pbase + c) * CE, CE)],
                    fbufs[b], fsems[b]).wait()

                def mul_row(e, cr, _gb=gbufs[b], _fb=fbufs[b]):
                    for j in range(MD // 16):
                        sl = pl.ds(j * 16, 16)
                        _gb[e, sl] = _gb[e, sl] * _fb[e, sl]
                    return cr

                lax.fori_loop(0, CE, mul_row, 0)
                pltpu.async_copy(gbufs[b], acc.at[srcb.at[c]], ssems[b],
                                 add=True)
            for b in range(2):
                c = 2 * i + b
                pltpu.make_async_copy(gbufs[b], acc.at[srcb.at[c]],
                                      ssems[b]).wait()

                @pl.when(c + 2 < HP)
                def _(b=b, c=c):
                    start(b, c + 2)

            return carry

        lax.fori_loop(0, HP // 2, body, 0)
        return pcarry

    lax.fori_loop(0, nph, phase, 0)

    plsc.subcore_barrier()
    pltpu.sync_copy(acc.at[pl.ds(sid * RPT, RPT)],
                    out_hbm.at[cid, pl.ds(sid * RPT, RPT)])


def _embed_body(x_ref, tab_ref, embw_ref, embb_ref, win0_ref, h_ref, hw_ref):
    xi = x_ref[...]
    fused = jnp.dot(tab_ref[...], embw_ref[...],
                    preferred_element_type=jnp.float32)
    oh = (lax.broadcasted_iota(jnp.int32, (BN, NT), 1) == xi).astype(jnp.float32)
    h0 = jax.nn.gelu(jnp.dot(oh, fused, preferred_element_type=jnp.float32)
                     + embb_ref[...])
    h_ref[...] = h0
    hw_ref[...] = jnp.dot(h0, win0_ref[...], preferred_element_type=jnp.float32)


_embed_call = pl.pallas_call(
    _embed_body,
    grid=(NSTEP,),
    in_specs=[
        pl.BlockSpec((BN, 1), lambda i: (i, 0)),
        pl.BlockSpec((NT, 5), lambda i: (0, 0)),
        pl.BlockSpec((5, D), lambda i: (0, 0)),
        pl.BlockSpec((1, D), lambda i: (0, 0)),
        pl.BlockSpec((D, MD), lambda i: (0, 0)),
    ],
    out_specs=[
        pl.BlockSpec((BN, D), lambda i: (i, 0)),
        pl.BlockSpec((BN, MD), lambda i: (i, 0)),
    ],
    out_shape=[
        jax.ShapeDtypeStruct((N, D), jnp.float32),
        jax.ShapeDtypeStruct((N, MD), jnp.float32),
    ],
)


def _filt_body(d2_ref, fw_ref, fb_ref, *out_refs):
    d2 = d2_ref[...]
    dist = jnp.sqrt(d2 + 1e-12)
    d = jnp.maximum(dist / CUTOFF, 1e-6)
    dsq = d * d
    d4 = dsq * dsq
    d5 = d4 * d
    a = -(P + 1) * (P + 2) / 2.0
    b = P * (P + 2)
    c = -P * (P + 1) / 2.0
    env = 1.0 / d + a * d5 + b * d5 * d + c * d5 * dsq
    env = jnp.where(dist < CUTOFF, env, 0.0) * np.float32(np.sqrt(2.0 / CUTOFF))
    freq = (lax.broadcasted_iota(jnp.int32, (1, NR), 1).astype(jnp.float32)
            + 1.0) * np.float32(np.pi)
    rbf = env * jnp.sin(d * freq)
    for blk in range(NB):
        out_refs[blk][...] = (
            jnp.dot(rbf, fw_ref[blk], preferred_element_type=jnp.float32)
            + fb_ref[blk])


_filt_call = pl.pallas_call(
    _filt_body,
    grid=(ESTEP,),
    in_specs=[
        pl.BlockSpec((BE, 1), lambda i: (i, 0)),
        pl.BlockSpec((NB, NR, MD), lambda i: (0, 0, 0)),
        pl.BlockSpec((NB, 1, MD), lambda i: (0, 0, 0)),
    ],
    out_specs=[pl.BlockSpec((BE, MD), lambda i: (i, 0)) for _ in range(NB)],
    out_shape=[jax.ShapeDtypeStruct((E_PAD, MD), jnp.float32)
               for _ in range(NB)],
)


def _block_body(parts_ref, h_ref, wout_ref, bout_ref, w1_ref, b1_ref,
                w2_ref, b2_ref, winn_ref, h_out_ref, hw_out_ref):
    agg = parts_ref[0] + parts_ref[1]
    t = jax.nn.gelu(jnp.dot(agg, wout_ref[...],
                            preferred_element_type=jnp.float32) + bout_ref[...])
    h1 = h_ref[...] + t
    u = jax.nn.gelu(jnp.dot(h1, w1_ref[...],
                            preferred_element_type=jnp.float32) + b1_ref[...])
    h2 = h1 + jnp.dot(u, w2_ref[...], preferred_element_type=jnp.float32) + b2_ref[...]
    h_out_ref[...] = h2
    hw_out_ref[...] = jnp.dot(h2, winn_ref[...], preferred_element_type=jnp.float32)


_block_call = pl.pallas_call(
    _block_body,
    grid=(NSTEP,),
    in_specs=[
        pl.BlockSpec((2, BN, MD), lambda i: (0, i, 0)),
        pl.BlockSpec((BN, D), lambda i: (i, 0)),
        pl.BlockSpec((MD, D), lambda i: (0, 0)),
        pl.BlockSpec((1, D), lambda i: (0, 0)),
        pl.BlockSpec((D, D), lambda i: (0, 0)),
        pl.BlockSpec((1, D), lambda i: (0, 0)),
        pl.BlockSpec((D, D), lambda i: (0, 0)),
        pl.BlockSpec((1, D), lambda i: (0, 0)),
        pl.BlockSpec((D, MD), lambda i: (0, 0)),
    ],
    out_specs=[
        pl.BlockSpec((BN, D), lambda i: (i, 0)),
        pl.BlockSpec((BN, MD), lambda i: (i, 0)),
    ],
    out_shape=[
        jax.ShapeDtypeStruct((N, D), jnp.float32),
        jax.ShapeDtypeStruct((N, MD), jnp.float32),
    ],
)


def _pool_body(b3_ref, h_ref, hw1_ref, hb1_ref, hw2_ref, hb2_ref, out_ref,
               gsum, cnt):
    i = pl.program_id(0)

    @pl.when(i == 0)
    def _():
        gsum[...] = jnp.zeros_like(gsum)
        cnt[...] = jnp.zeros_like(cnt)

    bb = jnp.minimum(b3_ref[0], G - 1)
    oh = (lax.broadcasted_iota(jnp.int32, (G, BN), 0) == bb).astype(jnp.float32)
    gsum[...] += jnp.dot(oh, h_ref[...], preferred_element_type=jnp.float32)
    cnt[...] += jnp.sum(oh, axis=1, keepdims=True)

    @pl.when(i == NSTEP - 1)
    def _():
        g = gsum[...] / jnp.maximum(cnt[...], 1.0)
        z = jax.nn.gelu(jnp.dot(g, hw1_ref[...],
                                preferred_element_type=jnp.float32) + hb1_ref[...])
        out_ref[...] = jnp.dot(z, hw2_ref[...],
                               preferred_element_type=jnp.float32) + hb2_ref[...]


_pool_call = pl.pallas_call(
    _pool_body,
    grid=(NSTEP,),
    in_specs=[
        pl.BlockSpec((1, 1, BN), lambda i: (i, 0, 0)),
        pl.BlockSpec((BN, D), lambda i: (i, 0)),
        pl.BlockSpec((D, D), lambda i: (0, 0)),
        pl.BlockSpec((1, D), lambda i: (0, 0)),
        pl.BlockSpec((D, OUT), lambda i: (0, 0)),
        pl.BlockSpec((1, OUT), lambda i: (0, 0)),
    ],
    out_specs=pl.BlockSpec((G, OUT), lambda i: (0, 0)),
    out_shape=jax.ShapeDtypeStruct((G, OUT), jnp.float32),
    scratch_shapes=[
        pltpu.VMEM((G, D), jnp.float32),
        pltpu.VMEM((G, 1), jnp.float32),
    ],
)


def kernel(x, pos, edge_index, batch, num_graphs, emb_table, emb_W, emb_b,
           W_in, filt_W, filt_b, W_out, b_out, fc_W1, fc_b1, fc_W2, fc_b2,
           head_W1, head_b1, head_W2, head_b2):
    del num_graphs
    pos_flat = pos.reshape(-1)
    src = edge_index[0].astype(jnp.int32)
    dst = edge_index[1].astype(jnp.int32)
    pad = E_PAD - E
    src_dist = jnp.pad(src, (0, pad)).reshape(NCHUNK, CE)
    dst_pad = jnp.pad(dst, (0, pad)).reshape(NCHUNK, CE)
    src_scat = jnp.pad(src, (0, pad), constant_values=N).reshape(NCHUNK, CE)

    d2 = _dist_kernel(pos_flat, src_dist, dst_pad)
    filts = _filt_call(d2.reshape(E_PAD, 1), filt_W, filt_b.reshape(NB, 1, MD))
    h, hw = _embed_call(x, emb_table, emb_W, emb_b.reshape(1, D), W_in[0])

    zeros = jnp.zeros((NPAD, MD), jnp.float32)
    for blk in range(NB):
        parts = _conv_kernel(hw, filts[blk], src_scat, dst_pad, zeros)
        h, hw = _block_call(parts, h,
                            W_out[blk], b_out[blk].reshape(1, D),
                            fc_W1[blk], fc_b1[blk].reshape(1, D),
                            fc_W2[blk], fc_b2[blk].reshape(1, D),
                            W_in[(blk + 1) % NB])

    return _pool_call(batch.reshape(NSTEP, 1, BN), h,
                      head_W1, head_b1.reshape(1, D),
                      head_W2, head_b2.reshape(1, OUT))
